# baseline (speedup 1.0000x reference)
import functools

import jax
import jax.numpy as jnp
from jax.experimental import pallas as pl

_LANES = 128


def _cmpx_row(v, t, group_desc=None):
    l, rows, _ = v.shape
    g = rows // (2 * t)
    v5 = v.reshape(l, g, 2, t, _LANES)
    a = v5[:, :, 0:1]
    b = v5[:, :, 1:2]
    mn = jnp.minimum(a, b)
    mx = jnp.maximum(a, b)
    if group_desc is None:
        lo, hi = mn, mx
    else:
        if group_desc.shape[1] == 1:
            gd = group_desc.reshape(group_desc.shape[0], 1, 1, 1, _LANES)
        else:
            gd = group_desc.reshape(group_desc.shape[0], g, 2, t, 1)[:, :, 0:1]
        lo = jnp.where(gd, mx, mn)
        hi = jnp.where(gd, mn, mx)
    return jnp.concatenate([lo, hi], axis=2).reshape(l, rows, _LANES)


def _row_iota(rows):
    return jax.lax.broadcasted_iota(jnp.int32, (rows, 1), 0)


def _lane_iota():
    return jax.lax.broadcasted_iota(jnp.int32, (1, _LANES), 1)


def _sort_chunks(v):
    desc = (jax.lax.broadcasted_iota(jnp.int32, (6, 1, 1), 0) % 3) != 0
    rowi = _row_iota(128)
    lanei = _lane_iota()
    vt = jnp.swapaxes(v, 1, 2)
    for k in range(1, 15):
        if k >= 8:
            v = jnp.swapaxes(vt, 1, 2)
            gd = (((rowi >> (k - 7)) & 1) == 1)[None] != desc
            for j in range(k - 1, 6, -1):
                v = _cmpx_row(v, 1 << (j - 7), gd)
            vt = jnp.swapaxes(v, 1, 2)
        if k < 7:
            gd = (((rowi >> k) & 1) == 1)[None] != desc
        else:
            gd = (((lanei >> (k - 7)) & 1) == 1)[None] != desc
        for j in range(min(k - 1, 6), -1, -1):
            vt = _cmpx_row(vt, 1 << j, gd)
    return vt


def _merge_asc(vt, log2n):
    v = _block_swap(vt)
    for j in range(log2n - 1, 6, -1):
        v = _cmpx_row(v, 1 << (j - 7))
    vt = _block_swap(v)
    for j in range(6, -1, -1):
        vt = _cmpx_row(vt, 1 << j)
    return vt


def _block_swap(v):
    l, rows, _ = v.shape
    nblk = rows // 128
    v4 = v.reshape(l, nblk, 128, _LANES)
    v4 = jnp.swapaxes(v4, 2, 3)
    return v4.reshape(l, rows, _LANES)


def _emd_body(x_ref, y_ref, o_ref, *, nreal):
    v = jnp.concatenate([x_ref[...], y_ref[...]], axis=0)
    v = v.reshape(2, 3, 128, _LANES).reshape(6, 128, _LANES)
    vt = _sort_chunks(v)
    vt = vt.reshape(2, 3, 128, _LANES)
    mt = vt[:, 0:2].reshape(2, 256, _LANES)
    mt = _merge_asc(mt, 15)
    infs = jnp.full((2, 1, 128, _LANES), jnp.inf, jnp.float32)
    wt = jnp.concatenate(
        [mt.reshape(2, 2, 128, _LANES), infs, vt[:, 2:3]], axis=1
    ).reshape(2, 512, _LANES)
    wt = _merge_asc(wt, 16)
    d = wt[0, :384] - wt[1, :384]
    o_ref[0] = jnp.full((8, _LANES), jnp.sum(d * d), jnp.float32)


def _emd_call(xp, yp, nreal):
    b = xp.shape[0]
    body = functools.partial(_emd_body, nreal=nreal)
    return pl.pallas_call(
        body,
        grid=(b,),
        in_specs=[
            pl.BlockSpec((1, 384, _LANES), lambda i: (i, 0, 0)),
            pl.BlockSpec((1, 384, _LANES), lambda i: (i, 0, 0)),
        ],
        out_specs=pl.BlockSpec((1, 8, _LANES), lambda i: (i, 0, 0)),
        out_shape=jax.ShapeDtypeStruct((b, 8, _LANES), jnp.float32),
    )(xp, yp)


def kernel(x, y):
    b = x.shape[0]
    n = x.shape[1] * x.shape[2]
    xp = x.reshape(b, 384, _LANES)
    yp = y.reshape(b, 384, _LANES)
    out = _emd_call(xp, yp, n)
    return jnp.sum(out[:, 0, 0]) / (b * n)


# speedup vs baseline: 1.0945x; 1.0945x over previous
import functools

import jax
import jax.numpy as jnp
from jax.experimental import pallas as pl

_LANES = 128


def _row_iota(rows):
    return jax.lax.broadcasted_iota(jnp.int32, (rows, 1), 0)


def _lane_iota():
    return jax.lax.broadcasted_iota(jnp.int32, (1, _LANES), 1)


def _cmpx_row(v, t, keep_min):
    l, rows, _ = v.shape
    v5 = v.reshape(l, rows // (2 * t), 2, t, _LANES)
    part = jnp.concatenate([v5[:, :, 1:2], v5[:, :, 0:1]], axis=2)
    part = part.reshape(l, rows, _LANES)
    mn = jnp.minimum(v, part)
    mx = jnp.maximum(v, part)
    return jnp.where(keep_min, mn, mx)


def _cmpx_lane(v, s, bj, keep_min):
    down = jnp.roll(v, -s, axis=2)
    up = jnp.roll(v, s, axis=2)
    part = jnp.where((bj == 0)[None], down, up)
    mn = jnp.minimum(v, part)
    mx = jnp.maximum(v, part)
    return jnp.where(keep_min, mn, mx)


def _block_swap(v):
    l, rows, _ = v.shape
    v4 = v.reshape(l, rows // 128, 128, _LANES)
    v4 = jnp.swapaxes(v4, 2, 3)
    return v4.reshape(l, rows, _LANES)


def _needs_natural(j):
    return j >= 10 or j <= 2


def _g_bit(b, frame_natural, rowi, lanei):
    if frame_natural:
        return ((rowi >> (b - 7)) & 1) if b >= 7 else ((lanei >> b) & 1)
    return ((lanei >> (b - 7)) & 1) if b >= 7 else ((rowi >> b) & 1)


def _substage(v, j, keep, bj, frame_natural):
    if frame_natural:
        if j >= 7:
            return _cmpx_row(v, 1 << (j - 7), keep)
        return _cmpx_lane(v, 1 << j, bj, keep)
    if j >= 7:
        return _cmpx_lane(v, 1 << (j - 7), bj, keep)
    return _cmpx_row(v, 1 << j, keep)


def _sort_chunks(v):
    desc = (jax.lax.broadcasted_iota(jnp.int32, (6, 1, 1), 0) % 3) != 0
    rowi = _row_iota(128)
    lanei = _lane_iota()
    natural = True
    for k in range(1, 15):
        for j in range(k - 1, -1, -1):
            if _needs_natural(j) != natural:
                v = _block_swap(v)
                natural = not natural
            bj = _g_bit(j, natural, rowi, lanei)
            bk = _g_bit(k, natural, rowi, lanei) if k < 14 else 0
            keep = (bj == bk)[None] != desc
            v = _substage(v, j, keep, bj, natural)
    if not natural:
        v = _block_swap(v)
    return v


def _merge_asc(v, log2n):
    l, rows, _ = v.shape
    rowi = _row_iota(rows)
    rowi_t = _row_iota(rows)
    lanei = _lane_iota()
    natural = True
    for j in range(log2n - 1, -1, -1):
        if _needs_natural(j) != natural:
            v = _block_swap(v)
            natural = not natural
        if natural and j >= 7:
            bj = (rowi >> (j - 7)) & 1
        else:
            bj = _g_bit(j, natural, rowi_t, lanei)
        keep = (bj == 0)[None]
        v = _substage(v, j, keep, bj, natural)
    if not natural:
        v = _block_swap(v)
    return v


def _emd_body(x_ref, y_ref, o_ref, *, nreal):
    v = jnp.concatenate([x_ref[...], y_ref[...]], axis=0)
    v = v.reshape(2, 3, 128, _LANES).reshape(6, 128, _LANES)
    v = _sort_chunks(v)
    v = v.reshape(2, 3, 128, _LANES)
    m = v[:, 0:2].reshape(2, 256, _LANES)
    m = _merge_asc(m, 15)
    infs = jnp.full((2, 1, 128, _LANES), jnp.inf, jnp.float32)
    w = jnp.concatenate(
        [m.reshape(2, 2, 128, _LANES), infs, v[:, 2:3]], axis=1
    ).reshape(2, 512, _LANES)
    w = _merge_asc(w, 16)
    d = w[0, :384] - w[1, :384]
    o_ref[0] = jnp.full((8, _LANES), jnp.sum(d * d), jnp.float32)


def _emd_call(xp, yp, nreal):
    b = xp.shape[0]
    body = functools.partial(_emd_body, nreal=nreal)
    return pl.pallas_call(
        body,
        grid=(b,),
        in_specs=[
            pl.BlockSpec((1, 384, _LANES), lambda i: (i, 0, 0)),
            pl.BlockSpec((1, 384, _LANES), lambda i: (i, 0, 0)),
        ],
        out_specs=pl.BlockSpec((1, 8, _LANES), lambda i: (i, 0, 0)),
        out_shape=jax.ShapeDtypeStruct((b, 8, _LANES), jnp.float32),
    )(xp, yp)


def kernel(x, y):
    b = x.shape[0]
    n = x.shape[1] * x.shape[2]
    xp = x.reshape(b, 384, _LANES)
    yp = y.reshape(b, 384, _LANES)
    out = _emd_call(xp, yp, n)
    return jnp.sum(out[:, 0, 0]) / (b * n)


# speedup vs baseline: 1.9621x; 1.7926x over previous
import functools

import jax
import jax.numpy as jnp
from jax.experimental import pallas as pl

_LANES = 128


def _row_iota(rows):
    return jax.lax.broadcasted_iota(jnp.int32, (rows, 1), 0)


def _lane_iota():
    return jax.lax.broadcasted_iota(jnp.int32, (1, _LANES), 1)


def _cmpx_row(v, t, keep_min):
    l, rows, _ = v.shape
    v5 = v.reshape(l, rows // (2 * t), 2, t, _LANES)
    part = jnp.concatenate([v5[:, :, 1:2], v5[:, :, 0:1]], axis=2)
    part = part.reshape(l, rows, _LANES)
    mn = jnp.minimum(v, part)
    mx = jnp.maximum(v, part)
    return jnp.where(keep_min, mn, mx)


def _cmpx_row_fine(v, t, keep_min):
    l, rows, _ = v.shape
    v4 = v.reshape(l, rows // 8, 8, _LANES)
    down = jnp.roll(v4, -t, axis=2).reshape(l, rows, _LANES)
    up = jnp.roll(v4, t, axis=2).reshape(l, rows, _LANES)
    bj = (_row_iota(rows) // t) & 1
    part = jnp.where((bj == 0)[None], down, up)
    mn = jnp.minimum(v, part)
    mx = jnp.maximum(v, part)
    return jnp.where(keep_min, mn, mx)


def _cmpx(v, t, keep_min):
    if t >= 8:
        return _cmpx_row(v, t, keep_min)
    return _cmpx_row_fine(v, t, keep_min)


def _block_swap(v):
    l, rows, _ = v.shape
    v4 = v.reshape(l, rows // 128, 128, _LANES)
    v4 = jnp.swapaxes(v4, 2, 3)
    return v4.reshape(l, rows, _LANES)


def _g_bit(b, natural, rowi, lanei):
    if natural:
        return ((rowi >> (b - 7)) & 1) if b >= 7 else ((lanei >> b) & 1)
    return ((lanei >> (b - 7)) & 1) if b >= 7 else ((rowi >> b) & 1)


def _sort_chunks(v):
    desc = (jax.lax.broadcasted_iota(jnp.int32, (6, 1, 1), 0) % 3) != 0
    rowi = _row_iota(128)
    lanei = _lane_iota()
    natural = True
    for k in range(1, 15):
        for j in range(k - 1, -1, -1):
            if (j >= 7) != natural:
                v = _block_swap(v)
                natural = not natural
            bj = _g_bit(j, natural, rowi, lanei)
            bk = _g_bit(k, natural, rowi, lanei) if k < 14 else 0
            keep = (bj == bk)[None] != desc
            v = _cmpx(v, 1 << (j - 7 if natural else j), keep)
    if not natural:
        v = _block_swap(v)
    return v


def _merge_asc(v, log2n):
    l, rows, _ = v.shape
    rowi = _row_iota(rows)
    lanei = _lane_iota()
    for j in range(log2n - 1, 6, -1):
        keep = (((rowi >> (j - 7)) & 1) == 0)[None]
        v = _cmpx(v, 1 << (j - 7), keep)
    v = _block_swap(v)
    for j in range(6, -1, -1):
        keep = (((rowi >> j) & 1) == 0)[None]
        v = _cmpx(v, 1 << j, keep)
    return _block_swap(v)


def _emd_body(x_ref, y_ref, o_ref, *, nreal):
    v = jnp.concatenate([x_ref[...], y_ref[...]], axis=0)
    v = v.reshape(2, 3, 128, _LANES).reshape(6, 128, _LANES)
    v = _sort_chunks(v)
    v = v.reshape(2, 3, 128, _LANES)
    m = v[:, 0:2].reshape(2, 256, _LANES)
    m = _merge_asc(m, 15)
    infs = jnp.full((2, 1, 128, _LANES), jnp.inf, jnp.float32)
    w = jnp.concatenate(
        [m.reshape(2, 2, 128, _LANES), infs, v[:, 2:3]], axis=1
    ).reshape(2, 512, _LANES)
    w = _merge_asc(w, 16)
    d = w[0, :384] - w[1, :384]
    o_ref[0] = jnp.full((8, _LANES), jnp.sum(d * d), jnp.float32)


def _emd_call(xp, yp, nreal):
    b = xp.shape[0]
    body = functools.partial(_emd_body, nreal=nreal)
    return pl.pallas_call(
        body,
        grid=(b,),
        in_specs=[
            pl.BlockSpec((1, 384, _LANES), lambda i: (i, 0, 0)),
            pl.BlockSpec((1, 384, _LANES), lambda i: (i, 0, 0)),
        ],
        out_specs=pl.BlockSpec((1, 8, _LANES), lambda i: (i, 0, 0)),
        out_shape=jax.ShapeDtypeStruct((b, 8, _LANES), jnp.float32),
    )(xp, yp)


def kernel(x, y):
    b = x.shape[0]
    n = x.shape[1] * x.shape[2]
    xp = x.reshape(b, 384, _LANES)
    yp = y.reshape(b, 384, _LANES)
    out = _emd_call(xp, yp, n)
    return jnp.sum(out[:, 0, 0]) / (b * n)
